# Initial kernel scaffold; baseline (speedup 1.0000x reference)
import jax
import jax.numpy as jnp
from jax.experimental import pallas as pl


def kernel(idx, weight):
    raise NotImplementedError("write your pallas kernel here")


# speedup vs baseline: 5.1140x; 5.1140x over previous
import functools

import jax
import jax.numpy as jnp
from jax import lax
from jax.experimental import pallas as pl
from jax.experimental.pallas import tpu as pltpu
from jax.experimental.pallas import tpu_sc as plsc

NUM_ROWS = 16384
SEQ = 200
VOCAB = 100000
DIM = 48

N = NUM_ROWS * SEQ
NC, NS = 2, 16
NW = NC * NS
PER_W = N // NW
SUB = 128
CHUNK = 1024
NSUB = CHUNK // SUB
NCHUNK = PER_W // CHUNK


def _emb_kernel(idx_hbm, table_hbm, out_hbm, idx_v, rows_v, sem):
  wid = lax.axis_index("s") * NC + lax.axis_index("c")
  sub_base = wid * (PER_W // SUB)
  row_base = wid * PER_W

  def body(g, carry):
    pltpu.sync_copy(idx_hbm.at[pl.ds(sub_base + g * NSUB, NSUB)], idx_v)
    copies = [
        pltpu.async_copy(
            table_hbm.at[idx_v.at[j]],
            rows_v.at[pl.ds(j * SUB, SUB)],
            sem,
        )
        for j in range(NSUB)
    ]
    for c in copies:
      c.wait()
    pltpu.sync_copy(rows_v, out_hbm.at[pl.ds(row_base + g * CHUNK, CHUNK)])
    return carry

  lax.fori_loop(0, NCHUNK, body, 0)


@jax.jit
def _emb(idx2d, weight):
  mesh = plsc.VectorSubcoreMesh(core_axis_name="c", subcore_axis_name="s")
  kfn = pl.kernel(
      _emb_kernel,
      out_type=jax.ShapeDtypeStruct((N, DIM), jnp.float32),
      mesh=mesh,
      scratch_types=[
          pltpu.VMEM((NSUB, SUB), jnp.int32),
          pltpu.VMEM((CHUNK, DIM), jnp.float32),
          pltpu.SemaphoreType.DMA,
      ],
      compiler_params=pltpu.CompilerParams(use_tc_tiling_on_sc=False),
  )
  return kfn(idx2d, weight)


def kernel(idx, weight):
  idx2d = idx.reshape(N // SUB, SUB)
  out = _emb(idx2d, weight)
  return out.reshape(NUM_ROWS, SEQ, DIM)


# speedup vs baseline: 5.3753x; 1.0511x over previous
import jax
import jax.numpy as jnp
from jax import lax
from jax.experimental import pallas as pl
from jax.experimental.pallas import tpu as pltpu
from jax.experimental.pallas import tpu_sc as plsc

NUM_ROWS = 16384
SEQ = 200
VOCAB = 100000
DIM = 48

N = NUM_ROWS * SEQ
NC, NS = 2, 16
NW = NC * NS
PER_W = N // NW
SUB = 128
CHUNK = 1024
NSUB = CHUNK // SUB
NCHUNK = PER_W // CHUNK
NBUF = 2


def _emb_kernel(idx_hbm, table_hbm, out_hbm, idx_v, rows_v, gsem, ssem):
  wid = lax.axis_index("s") * NC + lax.axis_index("c")
  sub_base = wid * (PER_W // SUB)
  row_base = wid * PER_W

  def load_and_fire(g, b):
    pltpu.sync_copy(idx_hbm.at[pl.ds(sub_base + g * NSUB, NSUB)], idx_v.at[b])
    for j in range(NSUB):
      pltpu.async_copy(
          table_hbm.at[idx_v.at[b, j]],
          rows_v.at[b, pl.ds(j * SUB, SUB)],
          gsem.at[b],
      )

  def drain_gathers(b):
    pltpu.make_async_copy(
        table_hbm.at[pl.ds(0, CHUNK)], rows_v.at[b], gsem.at[b]
    ).wait()

  def store(g, b):
    pltpu.async_copy(
        rows_v.at[b], out_hbm.at[pl.ds(row_base + g * CHUNK, CHUNK)], ssem.at[b]
    )

  def drain_store(b):
    pltpu.make_async_copy(
        rows_v.at[b], out_hbm.at[pl.ds(row_base, CHUNK)], ssem.at[b]
    ).wait()

  load_and_fire(0, 0)

  def pair(p, carry):
    for b in range(NBUF):
      g = p * NBUF + b
      nb = 1 - b

      @pl.when(g + 1 < NCHUNK)
      def _prefetch():
        @pl.when(g >= 1)
        def _wait_buf_free():
          drain_store(nb)

        load_and_fire(g + 1, nb)

      drain_gathers(b)
      store(g, b)
    return carry

  lax.fori_loop(0, NCHUNK // NBUF, pair, 0)
  drain_store(0)
  drain_store(1)


@jax.jit
def _emb(idx2d, weight):
  mesh = plsc.VectorSubcoreMesh(core_axis_name="c", subcore_axis_name="s")
  kfn = pl.kernel(
      _emb_kernel,
      out_type=jax.ShapeDtypeStruct((N, DIM), jnp.float32),
      mesh=mesh,
      scratch_types=[
          pltpu.VMEM((NBUF, NSUB, SUB), jnp.int32),
          pltpu.VMEM((NBUF, CHUNK, DIM), jnp.float32),
          pltpu.SemaphoreType.DMA((NBUF,)),
          pltpu.SemaphoreType.DMA((NBUF,)),
      ],
      compiler_params=pltpu.CompilerParams(use_tc_tiling_on_sc=False),
  )
  return kfn(idx2d, weight)


def kernel(idx, weight):
  idx2d = idx.reshape(N // SUB, SUB)
  out = _emb(idx2d, weight)
  return out.reshape(NUM_ROWS, SEQ, DIM)


# speedup vs baseline: 5.3830x; 1.0014x over previous
import jax
import jax.numpy as jnp
from jax import lax
from jax.experimental import pallas as pl
from jax.experimental.pallas import tpu as pltpu
from jax.experimental.pallas import tpu_sc as plsc

NUM_ROWS = 16384
SEQ = 200
VOCAB = 100000
DIM = 48

NC, NS = 2, 16
NW = NC * NS
ROWS_W = NUM_ROWS // NW
CROWS = 4
NCHUNK = ROWS_W // CROWS
NBUF = 3
SPLITS = ((0, 128), (128, SEQ - 128))


def _emb_kernel(idx_hbm, table_hbm, out_hbm, idx_v, rows_v, gsem, ssem):
  wid = lax.axis_index("s") * NC + lax.axis_index("c")
  row_base = wid * ROWS_W

  def load_and_fire(g, b):
    pltpu.sync_copy(idx_hbm.at[pl.ds(row_base + g * CROWS, CROWS)],
                    idx_v.at[b])
    for jr in range(CROWS):
      for off, sz in SPLITS:
        pltpu.async_copy(
            table_hbm.at[idx_v.at[b, jr, pl.ds(off, sz)]],
            rows_v.at[b, jr, pl.ds(off, sz)],
            gsem.at[b],
        )

  def drain_gathers(b):
    pltpu.make_async_copy(
        out_hbm.at[pl.ds(0, CROWS)], rows_v.at[b], gsem.at[b]
    ).wait()

  def store(g, b):
    pltpu.async_copy(
        rows_v.at[b], out_hbm.at[pl.ds(row_base + g * CROWS, CROWS)],
        ssem.at[b],
    )

  def drain_store(b):
    pltpu.make_async_copy(
        rows_v.at[b], out_hbm.at[pl.ds(row_base, CROWS)], ssem.at[b]
    ).wait()

  load_and_fire(0, 0)
  load_and_fire(1, 1)

  def body(g, carry):
    b = lax.rem(g, NBUF)
    nb = lax.rem(g + 2, NBUF)

    @pl.when(g >= 1)
    def _free_next_buffer():
      drain_store(nb)

    @pl.when(g + 2 < NCHUNK)
    def _prefetch():
      load_and_fire(g + 2, nb)

    drain_gathers(b)
    store(g, b)
    return carry

  lax.fori_loop(0, NCHUNK, body, 0)
  drain_store(lax.rem(NCHUNK - 1, NBUF))


@jax.jit
def _emb(idx, weight):
  mesh = plsc.VectorSubcoreMesh(core_axis_name="c", subcore_axis_name="s")
  kfn = pl.kernel(
      _emb_kernel,
      out_type=jax.ShapeDtypeStruct((NUM_ROWS, SEQ, DIM), jnp.float32),
      mesh=mesh,
      scratch_types=[
          pltpu.VMEM((NBUF, CROWS, SEQ), jnp.int32),
          pltpu.VMEM((NBUF, CROWS, SEQ, DIM), jnp.float32),
          pltpu.SemaphoreType.DMA((NBUF,)),
          pltpu.SemaphoreType.DMA((NBUF,)),
      ],
      compiler_params=pltpu.CompilerParams(use_tc_tiling_on_sc=False),
  )
  return kfn(idx, weight)


def kernel(idx, weight):
  return _emb(idx, weight)
